# baseline (speedup 1.0000x reference)
import functools

import jax
import jax.numpy as jnp
from jax import lax
from jax.experimental import pallas as pl
from jax.experimental.pallas import tpu as pltpu
from jax.experimental.pallas import tpu_sc as plsc

B = 16384
EMB = 32
IDX_CHUNK = 128


def _sc_gather(uta, utb, movie_table, uidx2, midx2):
    info = plsc.get_sparse_core_info()
    nc, ns = info.num_cores, info.num_subcores
    nw = nc * ns
    bpw = B // nw
    cpw = bpw // IDX_CHUNK

    mesh = plsc.VectorSubcoreMesh(core_axis_name="c", subcore_axis_name="s")

    @functools.partial(
        pl.kernel,
        mesh=mesh,
        out_type=(
            jax.ShapeDtypeStruct((B, 16), jnp.float32),
            jax.ShapeDtypeStruct((B, 16), jnp.float32),
            jax.ShapeDtypeStruct((B, EMB), jnp.float32),
        ),
        scratch_types=[
            pltpu.VMEM((cpw, IDX_CHUNK), jnp.int32),
            pltpu.VMEM((cpw, IDX_CHUNK), jnp.int32),
            pltpu.VMEM((bpw, 16), jnp.float32),
            pltpu.VMEM((bpw, 16), jnp.float32),
            pltpu.VMEM((bpw, EMB), jnp.float32),
            pltpu.SemaphoreType.DMA,
        ],
        compiler_params=pltpu.CompilerParams(use_tc_tiling_on_sc=False),
    )
    def gather_kernel(uta_hbm, utb_hbm, mt_hbm, uidx_hbm, midx_hbm,
                      ua_out, ub_out, m_out,
                      uidx_v, midx_v, ua_v, ub_v, mrows_v, sem):
        wid = lax.axis_index("s") * nc + lax.axis_index("c")
        base = wid * bpw
        pltpu.sync_copy(uidx_hbm.at[pl.ds(wid * cpw, cpw)], uidx_v)
        pltpu.sync_copy(midx_hbm.at[pl.ds(wid * cpw, cpw)], midx_v)
        copies = []
        for j in range(cpw):
            sl = pl.ds(j * IDX_CHUNK, IDX_CHUNK)
            copies.append(pltpu.async_copy(
                uta_hbm.at[uidx_v.at[j]], ua_v.at[sl], sem))
            copies.append(pltpu.async_copy(
                utb_hbm.at[uidx_v.at[j]], ub_v.at[sl], sem))
            copies.append(pltpu.async_copy(
                mt_hbm.at[midx_v.at[j]], mrows_v.at[sl], sem))
        for c in copies:
            c.wait()
        pltpu.sync_copy(ua_v, ua_out.at[pl.ds(base, bpw)])
        pltpu.sync_copy(ub_v, ub_out.at[pl.ds(base, bpw)])
        pltpu.sync_copy(mrows_v, m_out.at[pl.ds(base, bpw)])

    return gather_kernel(uta, utb, movie_table, uidx2, midx2)


def _mlp_body(ua_ref, ub_ref, m_ref, gao_ref, genres_ref, ctab_ref,
              w1_ref, b1_ref, w2_ref, b2_ref, w3_ref, b3_ref, out_ref):
    blk = ua_ref.shape[0]
    lanes = lax.broadcasted_iota(jnp.int32, (blk, 128), 1)
    gao = gao_ref[...]
    g = gao[:, 0:1]
    a = gao[:, 1:2]
    o = gao[:, 2:3]
    oh = ((lanes == g) | (lanes == (a + 2)) | (lanes == (o + 12)))
    demo = jnp.dot(oh.astype(jnp.float32), ctab_ref[...],
                   preferred_element_type=jnp.float32)
    h1 = (
        jnp.dot(ua_ref[...], w1_ref[0:16, :], preferred_element_type=jnp.float32)
        + jnp.dot(ub_ref[...], w1_ref[16:32, :], preferred_element_type=jnp.float32)
        + jnp.dot(m_ref[...], w1_ref[32:64, :], preferred_element_type=jnp.float32)
        + jnp.dot(demo, w1_ref[64:80, :], preferred_element_type=jnp.float32)
        + jnp.dot(genres_ref[...], w1_ref[80:98, :], preferred_element_type=jnp.float32)
        + b1_ref[...]
    )
    h1 = jnp.maximum(h1, 0.0)
    h2 = jnp.maximum(
        jnp.dot(h1, w2_ref[...], preferred_element_type=jnp.float32) + b2_ref[...],
        0.0)
    out_ref[...] = (jnp.dot(h2, w3_ref[...], preferred_element_type=jnp.float32)
                    + b3_ref[...])


def _tc_mlp(ua, ub, m, gao, genres, ctab, W1, b1, W2, b2, W3, b3, blk=2048):
    grid = B // blk
    full = lambda i: (0, 0)
    return pl.pallas_call(
        _mlp_body,
        grid=(grid,),
        in_specs=[
            pl.BlockSpec((blk, 16), lambda i: (i, 0)),
            pl.BlockSpec((blk, 16), lambda i: (i, 0)),
            pl.BlockSpec((blk, EMB), lambda i: (i, 0)),
            pl.BlockSpec((blk, 3), lambda i: (i, 0)),
            pl.BlockSpec((blk, 18), lambda i: (i, 0)),
            pl.BlockSpec((128, 16), full),
            pl.BlockSpec((98, 128), full),
            pl.BlockSpec((1, 128), full),
            pl.BlockSpec((128, 64), full),
            pl.BlockSpec((1, 64), full),
            pl.BlockSpec((64, 1), full),
            pl.BlockSpec((1, 1), full),
        ],
        out_specs=pl.BlockSpec((blk, 1), lambda i: (i, 0)),
        out_shape=jax.ShapeDtypeStruct((B, 1), jnp.float32),
    )(ua, ub, m, gao, genres, ctab, W1, b1, W2, b2, W3, b3)


def kernel(user, movie, gender, age, occupation, genres,
           user_table, movie_table, gender_table, age_table, occ_table,
           W1, b1, W2, b2, W3, b3):
    uidx2 = user.astype(jnp.int32).reshape(B // IDX_CHUNK, IDX_CHUNK)
    midx2 = movie.astype(jnp.int32).reshape(B // IDX_CHUNK, IDX_CHUNK)
    ua, ub, m = _sc_gather(user_table[:, 0:16], user_table[:, 16:32],
                           movie_table, uidx2, midx2)

    ctab = jnp.zeros((128, 16), jnp.float32)
    ctab = ctab.at[0:2, 0:4].set(gender_table)
    ctab = ctab.at[2:12, 4:8].set(age_table)
    ctab = ctab.at[12:37, 8:16].set(occ_table)

    gao = jnp.stack([gender.astype(jnp.int32), age.astype(jnp.int32),
                     occupation.astype(jnp.int32)], axis=1)

    out = _tc_mlp(ua, ub, m, gao, genres, ctab,
                  W1, b1.reshape(1, 128), W2, b2.reshape(1, 64),
                  W3, b3.reshape(1, 1))
    return jnp.squeeze(out, axis=1)


# speedup vs baseline: 2.2626x; 2.2626x over previous
import functools

import jax
import jax.numpy as jnp
from jax import lax
from jax.experimental import pallas as pl
from jax.experimental.pallas import tpu as pltpu
from jax.experimental.pallas import tpu_sc as plsc

B = 16384
EMB = 32
IDX_CHUNK = 128


def _sc_gather(user_table, movie_table, uidx2, midx2):
    info = plsc.get_sparse_core_info()
    nc, ns = info.num_cores, info.num_subcores
    nw = nc * ns
    bpw = B // nw
    cpw = bpw // IDX_CHUNK

    mesh = plsc.VectorSubcoreMesh(core_axis_name="c", subcore_axis_name="s")

    @functools.partial(
        pl.kernel,
        mesh=mesh,
        out_type=(
            jax.ShapeDtypeStruct((B, EMB), jnp.float32),
            jax.ShapeDtypeStruct((B, EMB), jnp.float32),
        ),
        scratch_types=[
            pltpu.VMEM((cpw, IDX_CHUNK), jnp.int32),
            pltpu.VMEM((cpw, IDX_CHUNK), jnp.int32),
            pltpu.VMEM((bpw, EMB), jnp.float32),
            pltpu.VMEM((bpw, EMB), jnp.float32),
            pltpu.SemaphoreType.DMA,
        ],
        compiler_params=pltpu.CompilerParams(use_tc_tiling_on_sc=False),
    )
    def gather_kernel(ut_hbm, mt_hbm, uidx_hbm, midx_hbm, u_out, m_out,
                      uidx_v, midx_v, urows_v, mrows_v, sem):
        wid = lax.axis_index("s") * nc + lax.axis_index("c")
        base = wid * bpw
        pltpu.sync_copy(uidx_hbm.at[pl.ds(wid * cpw, cpw)], uidx_v)
        pltpu.sync_copy(midx_hbm.at[pl.ds(wid * cpw, cpw)], midx_v)
        copies = []
        for j in range(cpw):
            copies.append(pltpu.async_copy(
                ut_hbm.at[uidx_v.at[j]],
                urows_v.at[pl.ds(j * IDX_CHUNK, IDX_CHUNK)], sem))
            copies.append(pltpu.async_copy(
                mt_hbm.at[midx_v.at[j]],
                mrows_v.at[pl.ds(j * IDX_CHUNK, IDX_CHUNK)], sem))
        for c in copies:
            c.wait()
        pltpu.sync_copy(urows_v, u_out.at[pl.ds(base, bpw)])
        pltpu.sync_copy(mrows_v, m_out.at[pl.ds(base, bpw)])

    return gather_kernel(user_table, movie_table, uidx2, midx2)


def _mlp_body(u_ref, m_ref, gao_ref, genres_ref, ctab_ref,
              w1_ref, b1_ref, w2_ref, b2_ref, w3_ref, b3_ref, out_ref):
    blk = u_ref.shape[0]
    lanes = lax.broadcasted_iota(jnp.int32, (blk, 128), 1)
    gao = gao_ref[...]
    g = gao[:, 0:1]
    a = gao[:, 1:2]
    o = gao[:, 2:3]
    oh = ((lanes == g) | (lanes == (a + 2)) | (lanes == (o + 12)))
    demo = jnp.dot(oh.astype(jnp.float32), ctab_ref[...],
                   preferred_element_type=jnp.float32)
    h1 = (
        jnp.dot(u_ref[...], w1_ref[0:32, :], preferred_element_type=jnp.float32)
        + jnp.dot(m_ref[...], w1_ref[32:64, :], preferred_element_type=jnp.float32)
        + jnp.dot(demo, w1_ref[64:80, :], preferred_element_type=jnp.float32)
        + jnp.dot(genres_ref[...], w1_ref[80:98, :], preferred_element_type=jnp.float32)
        + b1_ref[...]
    )
    h1 = jnp.maximum(h1, 0.0)
    h2 = jnp.maximum(
        jnp.dot(h1, w2_ref[...], preferred_element_type=jnp.float32) + b2_ref[...],
        0.0)
    out_ref[...] = (jnp.dot(h2, w3_ref[...], preferred_element_type=jnp.float32)
                    + b3_ref[...])


def _tc_mlp(u, m, gao, genres, ctab, W1, b1, W2, b2, W3, b3, blk=2048):
    grid = B // blk
    full = lambda i: (0, 0)
    return pl.pallas_call(
        _mlp_body,
        grid=(grid,),
        in_specs=[
            pl.BlockSpec((blk, EMB), lambda i: (i, 0)),
            pl.BlockSpec((blk, EMB), lambda i: (i, 0)),
            pl.BlockSpec((blk, 3), lambda i: (i, 0)),
            pl.BlockSpec((blk, 18), lambda i: (i, 0)),
            pl.BlockSpec((128, 16), full),
            pl.BlockSpec((98, 128), full),
            pl.BlockSpec((1, 128), full),
            pl.BlockSpec((128, 64), full),
            pl.BlockSpec((1, 64), full),
            pl.BlockSpec((64, 1), full),
            pl.BlockSpec((1, 1), full),
        ],
        out_specs=pl.BlockSpec((blk, 1), lambda i: (i, 0)),
        out_shape=jax.ShapeDtypeStruct((B, 1), jnp.float32),
    )(u, m, gao, genres, ctab, W1, b1, W2, b2, W3, b3)


def kernel(user, movie, gender, age, occupation, genres,
           user_table, movie_table, gender_table, age_table, occ_table,
           W1, b1, W2, b2, W3, b3):
    uidx2 = user.astype(jnp.int32).reshape(B // IDX_CHUNK, IDX_CHUNK)
    midx2 = movie.astype(jnp.int32).reshape(B // IDX_CHUNK, IDX_CHUNK)
    u, m = _sc_gather(user_table, movie_table, uidx2, midx2)

    ctab = jnp.zeros((128, 16), jnp.float32)
    ctab = ctab.at[0:2, 0:4].set(gender_table)
    ctab = ctab.at[2:12, 4:8].set(age_table)
    ctab = ctab.at[12:37, 8:16].set(occ_table)

    gao = jnp.stack([gender.astype(jnp.int32), age.astype(jnp.int32),
                     occupation.astype(jnp.int32)], axis=1)

    out = _tc_mlp(u, m, gao, genres, ctab,
                  W1, b1.reshape(1, 128), W2, b2.reshape(1, 64),
                  W3, b3.reshape(1, 1))
    return jnp.squeeze(out, axis=1)


# speedup vs baseline: 2.3090x; 1.0205x over previous
import functools

import jax
import jax.numpy as jnp
from jax import lax
from jax.experimental import pallas as pl
from jax.experimental.pallas import tpu as pltpu
from jax.experimental.pallas import tpu_sc as plsc

B = 16384
EMB = 32
IDX_CHUNK = 128


def _sc_gather(user_table, movie_table, uidx2, midx2):
    info = plsc.get_sparse_core_info()
    nc, ns = info.num_cores, info.num_subcores
    nw = nc * ns
    bpw = B // nw
    cpw = bpw // IDX_CHUNK

    mesh = plsc.VectorSubcoreMesh(core_axis_name="c", subcore_axis_name="s")

    @functools.partial(
        pl.kernel,
        mesh=mesh,
        out_type=(
            jax.ShapeDtypeStruct((B, 128), jnp.float32),
            jax.ShapeDtypeStruct((B, 128), jnp.float32),
        ),
        scratch_types=[
            pltpu.VMEM((cpw, IDX_CHUNK), jnp.int32),
            pltpu.VMEM((cpw, IDX_CHUNK), jnp.int32),
            pltpu.VMEM((bpw, EMB), jnp.float32),
            pltpu.VMEM((bpw, EMB), jnp.float32),
            pltpu.SemaphoreType.DMA,
        ],
        compiler_params=pltpu.CompilerParams(use_tc_tiling_on_sc=False),
    )
    def gather_kernel(ut_hbm, mt_hbm, uidx_hbm, midx_hbm, u_out, m_out,
                      uidx_v, midx_v, urows_v, mrows_v, sem):
        wid = lax.axis_index("s") * nc + lax.axis_index("c")
        base = wid * bpw
        pltpu.sync_copy(uidx_hbm.at[pl.ds(wid * cpw, cpw)], uidx_v)
        pltpu.sync_copy(midx_hbm.at[pl.ds(wid * cpw, cpw)], midx_v)
        copies = []
        for j in range(cpw):
            copies.append(pltpu.async_copy(
                ut_hbm.at[uidx_v.at[j]],
                urows_v.at[pl.ds(j * IDX_CHUNK, IDX_CHUNK)], sem))
            copies.append(pltpu.async_copy(
                mt_hbm.at[midx_v.at[j]],
                mrows_v.at[pl.ds(j * IDX_CHUNK, IDX_CHUNK)], sem))
        for c in copies:
            c.wait()
        pltpu.sync_copy(urows_v, u_out.at[pl.ds(base, bpw), pl.ds(0, EMB)])
        pltpu.sync_copy(mrows_v, m_out.at[pl.ds(base, bpw), pl.ds(0, EMB)])

    return gather_kernel(user_table, movie_table, uidx2, midx2)


def _mlp_body(u_ref, m_ref, gao_ref, genres_ref, ctab_ref,
              w1_ref, b1_ref, w2_ref, b2_ref, w3_ref, b3_ref, out_ref):
    blk = u_ref.shape[0]
    lanes = lax.broadcasted_iota(jnp.int32, (blk, 128), 1)
    gao = gao_ref[...]
    g = gao[:, 0:1]
    a = gao[:, 1:2]
    o = gao[:, 2:3]
    oh = ((lanes == g) | (lanes == (a + 2)) | (lanes == (o + 12)))
    demo = jnp.dot(oh.astype(jnp.float32), ctab_ref[...],
                   preferred_element_type=jnp.float32)
    h1 = (
        jnp.dot(u_ref[:, 0:EMB], w1_ref[0:32, :], preferred_element_type=jnp.float32)
        + jnp.dot(m_ref[:, 0:EMB], w1_ref[32:64, :], preferred_element_type=jnp.float32)
        + jnp.dot(demo, w1_ref[64:80, :], preferred_element_type=jnp.float32)
        + jnp.dot(genres_ref[...], w1_ref[80:98, :], preferred_element_type=jnp.float32)
        + b1_ref[...]
    )
    h1 = jnp.maximum(h1, 0.0)
    h2 = jnp.maximum(
        jnp.dot(h1, w2_ref[...], preferred_element_type=jnp.float32) + b2_ref[...],
        0.0)
    out_ref[...] = (jnp.dot(h2, w3_ref[...], preferred_element_type=jnp.float32)
                    + b3_ref[...])


def _tc_mlp(u, m, gao, genres, ctab, W1, b1, W2, b2, W3, b3, blk=2048):
    grid = B // blk
    full = lambda i: (0, 0)
    return pl.pallas_call(
        _mlp_body,
        grid=(grid,),
        in_specs=[
            pl.BlockSpec((blk, 128), lambda i: (i, 0)),
            pl.BlockSpec((blk, 128), lambda i: (i, 0)),
            pl.BlockSpec((blk, 3), lambda i: (i, 0)),
            pl.BlockSpec((blk, 18), lambda i: (i, 0)),
            pl.BlockSpec((128, 16), full),
            pl.BlockSpec((98, 128), full),
            pl.BlockSpec((1, 128), full),
            pl.BlockSpec((128, 64), full),
            pl.BlockSpec((1, 64), full),
            pl.BlockSpec((64, 1), full),
            pl.BlockSpec((1, 1), full),
        ],
        out_specs=pl.BlockSpec((blk, 1), lambda i: (i, 0)),
        out_shape=jax.ShapeDtypeStruct((B, 1), jnp.float32),
    )(u, m, gao, genres, ctab, W1, b1, W2, b2, W3, b3)


def kernel(user, movie, gender, age, occupation, genres,
           user_table, movie_table, gender_table, age_table, occ_table,
           W1, b1, W2, b2, W3, b3):
    uidx2 = user.astype(jnp.int32).reshape(B // IDX_CHUNK, IDX_CHUNK)
    midx2 = movie.astype(jnp.int32).reshape(B // IDX_CHUNK, IDX_CHUNK)
    u, m = _sc_gather(user_table, movie_table, uidx2, midx2)

    ctab = jnp.zeros((128, 16), jnp.float32)
    ctab = ctab.at[0:2, 0:4].set(gender_table)
    ctab = ctab.at[2:12, 4:8].set(age_table)
    ctab = ctab.at[12:37, 8:16].set(occ_table)

    gao = jnp.stack([gender.astype(jnp.int32), age.astype(jnp.int32),
                     occupation.astype(jnp.int32)], axis=1)

    out = _tc_mlp(u, m, gao, genres, ctab,
                  W1, b1.reshape(1, 128), W2, b2.reshape(1, 64),
                  W3, b3.reshape(1, 1))
    return jnp.squeeze(out, axis=1)
